# baseline (speedup 1.0000x reference)
import jax
import jax.numpy as jnp
from jax import lax
from jax.experimental import pallas as pl
from jax.experimental.pallas import tpu as pltpu
from jax.experimental.pallas import tpu_sc as plsc

_NC = 2
_NS = 16
_NW = _NC * _NS
_L = 16

_GE = 40
_GI = 2 * _GE
_CG = 10
_CE = _GE * _CG


def _proj_body(mvc_ref, w_ref, b_ref, out_ref):
    out_ref[...] = (
        jnp.dot(mvc_ref[...], w_ref[...], preferred_element_type=jnp.float32)
        + b_ref[...]
    )


def _node_projection(mvc, wcat, bcat):
    n, d = mvc.shape
    blk = 1000
    grid = n // blk
    return pl.pallas_call(
        _proj_body,
        grid=(grid,),
        in_specs=[
            pl.BlockSpec((blk, d), lambda i: (i, 0)),
            pl.BlockSpec((d, 2 * (wcat.shape[1] // 2)), lambda i: (0, 0)),
            pl.BlockSpec((1, wcat.shape[1]), lambda i: (0, 0)),
        ],
        out_specs=pl.BlockSpec((blk, wcat.shape[1]), lambda i: (i, 0)),
        out_shape=jax.ShapeDtypeStruct((n, wcat.shape[1]), jnp.float32),
    )(mvc, wcat, bcat)


def _edge_score_sc(table, idx3d, w2b, b2b, n_edges, hidden):
    ew = n_edges // _NW
    n_chunks = ew // _CE
    blocks = _CE // _L

    mesh = plsc.VectorSubcoreMesh(core_axis_name="c", subcore_axis_name="s")

    @pl.kernel(
        out_type=[
            jax.ShapeDtypeStruct((n_edges,), jnp.float32),
            jax.ShapeDtypeStruct((n_edges,), jnp.float32),
        ],
        mesh=mesh,
        compiler_params=pltpu.CompilerParams(
            use_tc_tiling_on_sc=False, needs_layout_passes=False
        ),
        scratch_types=[
            pltpu.VMEM((_CG, _GI), jnp.int32),
            pltpu.VMEM((2 * _CE, hidden), jnp.float32),
            pltpu.VMEM((2, _CE), jnp.float32),
            pltpu.VMEM((2, hidden, _L), jnp.float32),
            pltpu.VMEM((2, _L), jnp.float32),
            pltpu.SemaphoreType.DMA,
        ],
    )
    def k(t_hbm, idx_hbm, w2_hbm, b2_hbm, out0_hbm, out1_hbm, idx_v, s_v,
          o_v, w2_v, b2_v, sem):
        wid = lax.axis_index("s") * _NC + lax.axis_index("c")
        pltpu.sync_copy(w2_hbm, w2_v)
        pltpu.sync_copy(b2_hbm, b2_v)
        iota = lax.iota(jnp.int32, _L)
        rows0 = 2 * iota

        def chunk_body(c, _):
            base_e = wid * ew + c * _CE
            pltpu.sync_copy(idx_hbm.at[wid * n_chunks + c], idx_v)
            copies = []
            for g in range(_CG):
                copies.append(
                    pltpu.async_copy(
                        t_hbm.at[idx_v.at[g]],
                        s_v.at[pl.ds(g * _GI, _GI)],
                        sem,
                    )
                )
            for cp in copies:
                cp.wait()

            for b in range(blocks):
                o_v[0, pl.ds(b * _L, _L)] = b2_v[0, :]
                o_v[1, pl.ds(b * _L, _L)] = b2_v[1, :]

            def j_body(j, _):
                w0 = w2_v[0, j, :]
                w1 = w2_v[1, j, :]
                for b in range(blocks):
                    col_e = jnp.full((_L,), j + 2 * _L * hidden * b, jnp.int32)
                    p = plsc.load_gather(s_v, [rows0, col_e])
                    q = plsc.load_gather(s_v, [rows0, col_e + hidden])
                    r = jnp.maximum(p + q, 0.0)
                    plsc.addupdate(o_v.at[0, pl.ds(b * _L, _L)], r * w0)
                    plsc.addupdate(o_v.at[1, pl.ds(b * _L, _L)], r * w1)
                return 0

            lax.fori_loop(0, hidden, j_body, 0)
            pltpu.sync_copy(o_v.at[0], out0_hbm.at[pl.ds(base_e, _CE)])
            pltpu.sync_copy(o_v.at[1], out1_hbm.at[pl.ds(base_e, _CE)])
            return 0

        lax.fori_loop(0, n_chunks, chunk_body, 0)

    return k(table, idx3d, w2b, b2b)


def kernel(mvc, edge_index, slow_edge_mask, W1, b1, W2, b2):
    n_nodes, d_feat = mvc.shape
    n_edges = edge_index.shape[1]
    hidden = W1.shape[0]

    ei = jnp.where(~slow_edge_mask, edge_index, 0)

    offs = jnp.array([[0], [1]], dtype=jnp.int32)
    flat_idx = (2 * ei + offs).T.reshape(-1)
    idx3d = flat_idx.reshape(-1, _CG, _GI)

    wcat = jnp.concatenate([W1[:, :d_feat].T, W1[:, d_feat:].T], axis=1)
    bcat = jnp.concatenate([b1, jnp.zeros((hidden,), jnp.float32)])[None, :]
    r_nodes = _node_projection(mvc, wcat, bcat)
    table = r_nodes.reshape(2 * n_nodes, hidden)

    w2b = jnp.broadcast_to(W2[:, :, None], (2, hidden, _L)).astype(jnp.float32)
    b2b = jnp.broadcast_to(b2[:, None], (2, _L)).astype(jnp.float32)

    out0, out1 = _edge_score_sc(table, idx3d, w2b, b2b, n_edges, hidden)
    return jnp.stack([out0, out1], axis=1)


# speedup vs baseline: 3.2045x; 3.2045x over previous
import jax
import jax.numpy as jnp
from jax import lax
from jax.experimental import pallas as pl
from jax.experimental.pallas import tpu as pltpu
from jax.experimental.pallas import tpu_sc as plsc

_NC = 2
_NS = 16
_NW = _NC * _NS
_L = 16

_PADH = 65

_GE = 80
_CG = 5
_CE = _GE * _CG


def _proj_body(mvc_ref, w_ref, b_ref, out_ref):
    out_ref[...] = (
        jnp.dot(mvc_ref[...], w_ref[...], preferred_element_type=jnp.float32)
        + b_ref[...]
    )


def _node_projection(mvc, wcat, bcat):
    n, d = mvc.shape
    w = wcat.shape[1]
    blk = 1000
    return pl.pallas_call(
        _proj_body,
        grid=(n // blk,),
        in_specs=[
            pl.BlockSpec((blk, d), lambda i: (i, 0)),
            pl.BlockSpec((d, w), lambda i: (0, 0)),
            pl.BlockSpec((1, w), lambda i: (0, 0)),
        ],
        out_specs=pl.BlockSpec((blk, w), lambda i: (i, 0)),
        out_shape=jax.ShapeDtypeStruct((n, w), jnp.float32),
    )(mvc, wcat, bcat)


def _edge_score_sc(table, idxp, idxq, w2b, b2b, n_edges, hidden):
    ew = n_edges // _NW
    n_chunks = ew // _CE
    blocks = _CE // _L

    mesh = plsc.VectorSubcoreMesh(core_axis_name="c", subcore_axis_name="s")

    @pl.kernel(
        out_type=[
            jax.ShapeDtypeStruct((n_edges,), jnp.float32),
            jax.ShapeDtypeStruct((n_edges,), jnp.float32),
        ],
        mesh=mesh,
        compiler_params=pltpu.CompilerParams(
            use_tc_tiling_on_sc=False, needs_layout_passes=False
        ),
        scratch_types=[
            pltpu.VMEM((_CG, _GE), jnp.int32),
            pltpu.VMEM((_CG, _GE), jnp.int32),
            pltpu.VMEM((_CE, _PADH), jnp.float32),
            pltpu.VMEM((_CE, _PADH), jnp.float32),
            pltpu.VMEM((2, _CE), jnp.float32),
            pltpu.VMEM((2, hidden, _L), jnp.float32),
            pltpu.VMEM((2, _L), jnp.float32),
            pltpu.SemaphoreType.DMA,
        ],
    )
    def k(t_hbm, idxp_hbm, idxq_hbm, w2_hbm, b2_hbm, out0_hbm, out1_hbm,
          ip_v, iq_v, sp_v, sq_v, o_v, w2_v, b2_v, sem):
        wid = lax.axis_index("s") * _NC + lax.axis_index("c")
        pltpu.sync_copy(w2_hbm, w2_v)
        pltpu.sync_copy(b2_hbm, b2_v)
        iota = lax.iota(jnp.int32, _L)

        def chunk_body(c, _):
            base_e = wid * ew + c * _CE
            pltpu.sync_copy(idxp_hbm.at[wid * n_chunks + c], ip_v)
            pltpu.sync_copy(idxq_hbm.at[wid * n_chunks + c], iq_v)
            copies = []
            for g in range(_CG):
                copies.append(
                    pltpu.async_copy(
                        t_hbm.at[ip_v.at[g]],
                        sp_v.at[pl.ds(g * _GE, _GE)],
                        sem,
                    )
                )
                copies.append(
                    pltpu.async_copy(
                        t_hbm.at[iq_v.at[g]],
                        sq_v.at[pl.ds(g * _GE, _GE)],
                        sem,
                    )
                )
            for cp in copies:
                cp.wait()

            @plsc.parallel_loop(0, blocks, 1, unroll=2)
            def block_body(b):
                rows = _L * b + iota
                acc = [b2_v[0, :], jnp.zeros((_L,), jnp.float32),
                       b2_v[1, :], jnp.zeros((_L,), jnp.float32)]
                for j in range(hidden):
                    col = jnp.full((_L,), j, jnp.int32)
                    p = plsc.load_gather(sp_v, [rows, col])
                    q = plsc.load_gather(sq_v, [rows, col])
                    r = jnp.maximum(p + q, 0.0)
                    par = j & 1
                    acc[par] = acc[par] + r * w2_v[0, j, :]
                    acc[2 + par] = acc[2 + par] + r * w2_v[1, j, :]
                o_v[0, pl.ds(b * _L, _L)] = acc[0] + acc[1]
                o_v[1, pl.ds(b * _L, _L)] = acc[2] + acc[3]

            del block_body
            pltpu.sync_copy(o_v.at[0], out0_hbm.at[pl.ds(base_e, _CE)])
            pltpu.sync_copy(o_v.at[1], out1_hbm.at[pl.ds(base_e, _CE)])
            return 0

        lax.fori_loop(0, n_chunks, chunk_body, 0)

    return k(table, idxp, idxq, w2b, b2b)


def kernel(mvc, edge_index, slow_edge_mask, W1, b1, W2, b2):
    n_nodes, d_feat = mvc.shape
    n_edges = edge_index.shape[1]
    hidden = W1.shape[0]
    pad = _PADH - hidden

    ei = jnp.where(~slow_edge_mask, edge_index, 0)

    n_tot = n_edges // _CE
    idxp = (2 * ei[0]).reshape(n_tot, _CG, _GE)
    idxq = (2 * ei[1] + 1).reshape(n_tot, _CG, _GE)

    zcol = jnp.zeros((d_feat, pad), jnp.float32)
    wcat = jnp.concatenate(
        [W1[:, :d_feat].T, zcol, W1[:, d_feat:].T, zcol], axis=1)
    bcat = jnp.concatenate(
        [b1, jnp.zeros((_PADH + pad,), jnp.float32)])[None, :]
    r_nodes = _node_projection(mvc, wcat, bcat)
    table = r_nodes.reshape(2 * n_nodes, _PADH)

    w2b = jnp.broadcast_to(W2[:, :, None], (2, hidden, _L)).astype(jnp.float32)
    b2b = jnp.broadcast_to(b2[:, None], (2, _L)).astype(jnp.float32)

    out0, out1 = _edge_score_sc(table, idxp, idxq, w2b, b2b, n_edges, hidden)
    return jnp.stack([out0, out1], axis=1)
